# Initial kernel scaffold; baseline (speedup 1.0000x reference)
import jax
import jax.numpy as jnp
from jax.experimental import pallas as pl


def kernel(center_index, context_indices, W_emb, W_out):
    raise NotImplementedError("write your pallas kernel here")


# speedup vs baseline: 1.8179x; 1.8179x over previous
import functools

import jax
import jax.numpy as jnp
from jax import lax
from jax.experimental import pallas as pl
from jax.experimental.pallas import tpu as pltpu
from jax.experimental.pallas import tpu_sc as plsc

_TILE_V = 2048
_NEG = -3e38


@functools.lru_cache(maxsize=None)
def _make_sc_gather(B, Wn, V, D):
    info = plsc.get_sparse_core_info()
    NC, NS = info.num_cores, info.num_subcores
    NW = NC * NS
    assert B % NW == 0
    b_per_w = B // NW
    c_per_w = (B * Wn) // NW
    CH = 128
    assert c_per_w % CH == 0 and CH % Wn == 0
    n_ch = c_per_w // CH
    bpc = CH // Wn
    assert D % 16 == 0
    nl = D // 16

    mesh = plsc.VectorSubcoreMesh(core_axis_name="c", subcore_axis_name="s")

    scratch = [pltpu.VMEM((b_per_w,), jnp.int32),
               pltpu.VMEM((b_per_w, D), jnp.float32)]
    scratch += [pltpu.VMEM((CH,), jnp.int32) for _ in range(n_ch)]
    scratch += [pltpu.VMEM((CH, D), jnp.float32) for _ in range(n_ch)]
    scratch += [pltpu.VMEM((b_per_w, D), jnp.float32),
                pltpu.SemaphoreType.DMA]

    @functools.partial(
        pl.kernel,
        mesh=mesh,
        out_type=(jax.ShapeDtypeStruct((B, D), jnp.float32),
                  jax.ShapeDtypeStruct((B, D), jnp.float32)),
        scratch_types=scratch,
        compiler_params=pltpu.CompilerParams(use_tc_tiling_on_sc=False),
    )
    def sc_gather(cidx_hbm, ctx_hbm, wemb_hbm, wout_hbm, out_h, out_csum,
                  idx_h, rows_h, *rest):
        idx_c = rest[0:n_ch]
        rows_c = rest[n_ch:2 * n_ch]
        csum_v = rest[2 * n_ch]
        sem = rest[2 * n_ch + 1]

        wid = lax.axis_index("s") * NC + lax.axis_index("c")
        hbase = wid * b_per_w
        cbase = wid * c_per_w

        pltpu.sync_copy(cidx_hbm.at[pl.ds(hbase, b_per_w)], idx_h)
        for ch in range(n_ch):
            pltpu.sync_copy(ctx_hbm.at[pl.ds(cbase + ch * CH, CH)], idx_c[ch])

        cps = [pltpu.async_copy(wemb_hbm.at[idx_h], rows_h, sem)]
        for ch in range(n_ch):
            cps.append(pltpu.async_copy(wout_hbm.at[idx_c[ch]], rows_c[ch], sem))
        for cp in cps:
            cp.wait()

        for ch in range(n_ch):
            rc = rows_c[ch]

            def wbody(i, carry, rc=rc, off=ch * bpc):
                for l in range(nl):
                    acc = rc[i * Wn, pl.ds(l * 16, 16)]
                    for w in range(1, Wn):
                        acc = acc + rc[i * Wn + w, pl.ds(l * 16, 16)]
                    csum_v[off + i, pl.ds(l * 16, 16)] = acc
                return carry

            lax.fori_loop(0, bpc, wbody, 0)

        pltpu.sync_copy(rows_h, out_h.at[pl.ds(hbase, b_per_w)])
        pltpu.sync_copy(csum_v, out_csum.at[pl.ds(hbase, b_per_w)])

    return sc_gather


def _tc_body(h_ref, csum_ref, wout_ref, out_ref, m_ref, s_ref, *, B, Wn, V,
             tile_v):
    i = pl.program_id(0)
    nt = pl.num_programs(0)

    @pl.when(i == 0)
    def _init():
        m_ref[...] = jnp.full(m_ref.shape, _NEG, jnp.float32)
        s_ref[...] = jnp.zeros(s_ref.shape, jnp.float32)

    hb = h_ref[...].astype(jnp.bfloat16)
    wb = wout_ref[...].astype(jnp.bfloat16)
    logits = lax.dot_general(hb, wb, (((1,), (1,)), ((), ())),
                             preferred_element_type=jnp.float32)
    col = i * tile_v + lax.broadcasted_iota(jnp.int32, (1, tile_v), 1)
    logits = jnp.where(col < V, logits, _NEG)

    m_old = m_ref[...]
    m_new = jnp.maximum(m_old, jnp.max(logits, axis=1, keepdims=True))
    s_ref[...] = (s_ref[...] * jnp.exp(m_old - m_new)
                  + jnp.sum(jnp.exp(logits - m_new), axis=1, keepdims=True))
    m_ref[...] = m_new

    @pl.when(i == nt - 1)
    def _fin():
        lse = m_ref[...] + jnp.log(s_ref[...])
        ctx_total = jnp.sum(h_ref[...] * csum_ref[...])
        out_ref[0, 0] = jnp.sum(lse) / B - ctx_total / (B * Wn)


@functools.lru_cache(maxsize=None)
def _make_tc_loss(B, Wn, V, D):
    tile_v = _TILE_V
    nt = pl.cdiv(V, tile_v)
    body = functools.partial(_tc_body, B=B, Wn=Wn, V=V, tile_v=tile_v)
    return pl.pallas_call(
        body,
        grid=(nt,),
        in_specs=[
            pl.BlockSpec((B, D), lambda i: (0, 0)),
            pl.BlockSpec((B, D), lambda i: (0, 0)),
            pl.BlockSpec((tile_v, D), lambda i: (i, 0)),
        ],
        out_specs=pl.BlockSpec((1, 1), lambda i: (0, 0),
                               memory_space=pltpu.SMEM),
        out_shape=jax.ShapeDtypeStruct((1, 1), jnp.float32),
        scratch_shapes=[
            pltpu.VMEM((B, 1), jnp.float32),
            pltpu.VMEM((B, 1), jnp.float32),
        ],
    )


def kernel(center_index, context_indices, W_emb, W_out):
    B, Wn = context_indices.shape
    V, D = W_emb.shape
    cidx = center_index.astype(jnp.int32)
    ctx = context_indices.astype(jnp.int32).reshape(-1)
    h, csum = _make_sc_gather(B, Wn, V, D)(cidx, ctx, W_emb, W_out)
    loss = _make_tc_loss(B, Wn, V, D)(h, csum, W_out)
    return loss[0, 0]
